# Initial kernel scaffold; baseline (speedup 1.0000x reference)
import jax
import jax.numpy as jnp
from jax.experimental import pallas as pl


def kernel(x, edge_index, W1, b1, W2, b2, W3, b3, W4, b4, W5, b5, Wl1, bl1, Wl2, bl2):
    raise NotImplementedError("write your pallas kernel here")


# speedup vs baseline: 1.7323x; 1.7323x over previous
import functools

import jax
import jax.numpy as jnp
from jax.experimental import pallas as pl
from jax.experimental.pallas import tpu as pltpu

_N = 50000
_E = 800000
_BLK = 128
_NPAD = 50048


def _mlp_body(feat_ref, x_ref, wl1_ref, bl1_ref, wl2_ref, bl2_ref, out_ref):
    h = jnp.maximum(
        jnp.dot(feat_ref[...], wl1_ref[...], preferred_element_type=jnp.float32)
        + bl1_ref[...],
        0.0,
    )
    out_ref[...] = (
        x_ref[...]
        + jnp.dot(h, wl2_ref[...], preferred_element_type=jnp.float32)
        + bl2_ref[...]
    )


@jax.jit
def _final_mlp(feat, xpad, Wl1, bl1, Wl2p, bl2p):
    grid = (_NPAD // _BLK,)
    return pl.pallas_call(
        _mlp_body,
        grid=grid,
        in_specs=[
            pl.BlockSpec((_BLK, 128), lambda i: (i, 0)),
            pl.BlockSpec((_BLK, 128), lambda i: (i, 0)),
            pl.BlockSpec((128, 128), lambda i: (0, 0)),
            pl.BlockSpec((1, 128), lambda i: (0, 0)),
            pl.BlockSpec((128, 128), lambda i: (0, 0)),
            pl.BlockSpec((1, 128), lambda i: (0, 0)),
        ],
        out_specs=pl.BlockSpec((_BLK, 128), lambda i: (i, 0)),
        out_shape=jax.ShapeDtypeStruct((_NPAD, 128), jnp.float32),
    )(feat, xpad, Wl1, bl1, Wl2p, bl2p)


def _layer(x, src, dst, W, b):
    fi = x.shape[1]
    A = x @ (W[:fi] - W[fi:]) + b
    B = x @ W[fi:]
    M = jax.ops.segment_max(B[src], dst, num_segments=_N)
    has = ~jnp.isneginf(M[:, :1])
    return jnp.where(has, jnp.maximum(A + M, 0.0), 0.0)


def kernel(x, edge_index, W1, b1, W2, b2, W3, b3, W4, b4, W5, b5,
           Wl1, bl1, Wl2, bl2):
    src = edge_index[0]
    dst = edge_index[1]
    x1 = _layer(x, src, dst, W1, b1)
    x2 = _layer(x1, src, dst, W2, b2)
    x3 = _layer(x2, src, dst, W3, b3)
    x4 = _layer(x3, src, dst, W4, b4)
    x5 = _layer(x4, src, dst, W5, b5)
    g = jnp.max(x5, axis=0, keepdims=True)
    feat = jnp.concatenate([x5, jnp.broadcast_to(g, (_N, 64))], axis=1)
    featp = jnp.zeros((_NPAD, 128), jnp.float32).at[:_N].set(feat)
    xpad = jnp.zeros((_NPAD, 128), jnp.float32).at[:_N, :3].set(x)
    Wl2p = jnp.zeros((128, 128), jnp.float32).at[:, :3].set(Wl2)
    bl2p = jnp.zeros((1, 128), jnp.float32).at[0, :3].set(bl2)
    out = _final_mlp(featp, xpad, Wl1, bl1.reshape(1, 128), Wl2p, bl2p)
    return out[:_N, :3]


# speedup vs baseline: 3.7197x; 2.1472x over previous
import functools

import jax
import jax.numpy as jnp
from jax import lax
from jax.experimental import pallas as pl
from jax.experimental.pallas import tpu as pltpu
from jax.experimental.pallas import tpu_sc as plsc

_N = 50000
_E = 800000
_NW = 32
_RPW = 1600
_NPAD = _NW * _RPW
_NEG = -3.0e38
_C = 1600
_NCHUNK = _E // _C
_VPC = _C // 16
_F = 512
_ECAP = (_E // _F + 2) * _F
_G = 128
_HF = 32
_ACC_ROWS = _RPW + 2

_BLK = 128
_NBLK = _NPAD // _BLK

_sc_mesh = plsc.VectorSubcoreMesh(core_axis_name="c", subcore_axis_name="s")


@functools.partial(
    pl.kernel,
    out_type=[
        jax.ShapeDtypeStruct((_NW * _ECAP,), jnp.int32),
        jax.ShapeDtypeStruct((_NW * _ECAP,), jnp.int32),
        jax.ShapeDtypeStruct((_NW * 16,), jnp.int32),
    ],
    mesh=_sc_mesh,
    scratch_types=[
        pltpu.VMEM((_C,), jnp.int32),
        pltpu.VMEM((_C,), jnp.int32),
        pltpu.VMEM((_C,), jnp.int32),
        pltpu.VMEM((_C,), jnp.int32),
        pltpu.VMEM((_F + _C + 16,), jnp.int32),
        pltpu.VMEM((_F + _C + 16,), jnp.int32),
        pltpu.VMEM((16,), jnp.int32),
        pltpu.SemaphoreType.DMA,
        pltpu.SemaphoreType.DMA,
        pltpu.SemaphoreType.DMA,
        pltpu.SemaphoreType.DMA,
        pltpu.SemaphoreType.DMA,
    ],
    compiler_params=pltpu.CompilerParams(needs_layout_passes=False),
)
def _bucketize(src_hbm, dst_hbm, lsrc_hbm, lld_hbm, cnt_hbm,
               dstb0, dstb1, srcb0, srcb1, sbs, sbl, cntv,
               sd0, sd1, ss0, ss1, so):
    cidx = lax.axis_index("c")
    sidx = lax.axis_index("s")
    wid = sidx * 2 + cidx
    lo = wid * _RPW
    lanes = lax.iota(jnp.int32, 16)
    pad16 = jnp.full((16,), _RPW, jnp.int32)

    def _issue(k, dref, sref, dsem, ssem):
        pltpu.make_async_copy(dst_hbm.at[pl.ds(k * _C, _C)], dref, dsem).start()
        pltpu.make_async_copy(src_hbm.at[pl.ds(k * _C, _C)], sref, ssem).start()

    def _wait(k, dref, sref, dsem, ssem):
        pltpu.make_async_copy(dst_hbm.at[pl.ds(k * _C, _C)], dref, dsem).wait()
        pltpu.make_async_copy(src_hbm.at[pl.ds(k * _C, _C)], sref, ssem).wait()

    def _scan(dref, sref, pending):
        def scan_body(v, pending):
            d = dref[pl.ds(v * 16, 16)]
            s = sref[pl.ds(v * 16, 16)]
            m = (d >= lo) & (d < lo + _RPW)
            cum = plsc.cumsum(m.astype(jnp.int32))
            pos = pending + cum - 1
            plsc.store_scatter(sbs, [pos], s, mask=m)
            plsc.store_scatter(sbl, [pos], d - lo, mask=m)
            return pending + cum[15]

        return lax.fori_loop(0, _VPC, scan_body, pending, unroll=2)

    def _flush(state):
        pending, flushed = state
        cp = pltpu.make_async_copy(
            sbs.at[pl.ds(0, _F)],
            lsrc_hbm.at[pl.ds(pl.multiple_of(wid * _ECAP + flushed, 8), _F)],
            so)
        cp.start()
        cp.wait()
        cp = pltpu.make_async_copy(
            sbl.at[pl.ds(0, _F)],
            lld_hbm.at[pl.ds(pl.multiple_of(wid * _ECAP + flushed, 8), _F)],
            so)
        cp.start()
        cp.wait()

        def shift(v, carry):
            sbs[pl.ds(v * 16, 16)] = sbs[pl.ds(_F + v * 16, 16)]
            sbl[pl.ds(v * 16, 16)] = sbl[pl.ds(_F + v * 16, 16)]
            return carry

        lax.fori_loop(0, (_C + 16) // 16, shift, 0, unroll=2)
        return pending - _F, flushed + _F

    def chunk_pair(i, state):
        k0 = i * 2
        state = _drain_scan(k0, dstb0, srcb0, sd0, ss0, state)

        @pl.when(k0 + 2 < _NCHUNK)
        def _():
            _issue(k0 + 2, dstb0, srcb0, sd0, ss0)

        state = _drain_scan(k0 + 1, dstb1, srcb1, sd1, ss1, state)

        @pl.when(k0 + 3 < _NCHUNK)
        def _():
            _issue(k0 + 3, dstb1, srcb1, sd1, ss1)

        return state

    def _drain_scan(k, dref, sref, dsem, ssem, state):
        pending, flushed = state
        _wait(k, dref, sref, dsem, ssem)
        pending = _scan(dref, sref, pending)
        return lax.while_loop(
            lambda st: st[0] >= _F, _flush, (pending, flushed))

    _issue(0, dstb0, srcb0, sd0, ss0)
    _issue(1, dstb1, srcb1, sd1, ss1)
    pending, flushed = lax.fori_loop(
        0, _NCHUNK // 2, chunk_pair, (jnp.int32(0), jnp.int32(0)))

    for j in range(_F // 16):
        pos = pending + lanes + j * 16
        plsc.store_scatter(sbs, [pos], lo + lanes + j * 16)
        plsc.store_scatter(sbl, [pos], pad16)
    _flush((pending, flushed))
    total = flushed + _F

    cntv[...] = jnp.where(lanes == 0, total, 0)
    pltpu.sync_copy(cntv, cnt_hbm.at[pl.ds(pl.multiple_of(wid * 16, 8), 16)])


@functools.partial(
    pl.kernel,
    out_type=jax.ShapeDtypeStruct((_NPAD * _HF,), jnp.float32),
    mesh=_sc_mesh,
    scratch_types=[
        pltpu.VMEM((_G,), jnp.int32),
        pltpu.VMEM((_G,), jnp.int32),
        pltpu.VMEM((_G, _HF), jnp.float32),
        pltpu.VMEM((_ACC_ROWS * _HF,), jnp.float32),
        pltpu.VMEM((16,), jnp.int32),
        pltpu.SemaphoreType.DMA,
        pltpu.SemaphoreType.DMA,
        pltpu.SemaphoreType.DMA,
        pltpu.SemaphoreType.DMA,
    ],
    compiler_params=pltpu.CompilerParams(use_tc_tiling_on_sc=False),
)
def _segmax_half(bh_hbm, lsrc_hbm, lld_hbm, cnt_hbm, mh_hbm,
                 gidx, lldv, rows, acc, cntv,
                 sg, sl0, sl1, sc):
    cidx = lax.axis_index("c")
    sidx = lax.axis_index("s")
    wid = sidx * 2 + cidx
    lo = wid * _RPW
    neg16 = jnp.full((16,), _NEG, jnp.float32)

    def _init(i, carry):
        acc[pl.ds(i * 16, 16)] = neg16
        return carry

    lax.fori_loop(0, _ACC_ROWS * _HF // 16, _init, 0, unroll=4)

    cp = pltpu.make_async_copy(cnt_hbm.at[pl.ds(pl.multiple_of(wid * 16, 8), 16)], cntv, sc)
    cp.start()
    cp.wait()
    n = cntv[...][0]
    nb = n // _G

    def batch_body(b, carry):
        cp = pltpu.make_async_copy(
            lsrc_hbm.at[pl.ds(pl.multiple_of(wid * _ECAP + b * _G, 8), _G)], gidx, sl0)
        cp.start()
        cp2 = pltpu.make_async_copy(
            lld_hbm.at[pl.ds(pl.multiple_of(wid * _ECAP + b * _G, 8), _G)], lldv, sl1)
        cp2.start()
        cp.wait()
        cp2.wait()
        cpg = pltpu.make_async_copy(bh_hbm.at[gidx], rows, sg)
        cpg.start()
        cpg.wait()
        for q in range(_G // 16):
            ldv = lldv[pl.ds(q * 16, 16)]
            for lane in range(16):
                base = pl.multiple_of(ldv[lane] * _HF, 8)
                for cg in range(_HF // 16):
                    cur = acc[pl.ds(base + cg * 16, 16)]
                    new = rows[q * 16 + lane, pl.ds(cg * 16, 16)]
                    acc[pl.ds(base + cg * 16, 16)] = jnp.maximum(cur, new)
        return carry

    lax.fori_loop(0, nb, batch_body, 0)

    pltpu.sync_copy(acc.at[pl.ds(0, _RPW * _HF)],
                    mh_hbm.at[pl.ds(pl.multiple_of(lo * _HF, 8), _RPW * _HF)])


def _segmax(b0, b1, lsrc, lld, cnt):
    m0 = _segmax_half(b0, lsrc, lld, cnt).reshape(_NPAD, _HF)
    m1 = _segmax_half(b1, lsrc, lld, cnt).reshape(_NPAD, _HF)
    return m0, m1


def _ab_body(x_ref, wt_ref, wb_ref, b_ref, a_ref, b0_ref, b1_ref):
    xb = x_ref[...]
    wb = wb_ref[...]
    wa = wt_ref[...] - wb
    a_ref[...] = (
        jnp.dot(xb, wa, preferred_element_type=jnp.float32) + b_ref[...]
    )
    bb = jnp.dot(xb, wb, preferred_element_type=jnp.float32)
    b0_ref[...] = bb[:, :_HF]
    b1_ref[...] = bb[:, _HF:]


def _ab_layer1(x8, w1t, w1b, b1):
    return pl.pallas_call(
        _ab_body,
        grid=(_NBLK,),
        in_specs=[
            pl.BlockSpec((_BLK, 8), lambda i: (i, 0)),
            pl.BlockSpec((8, 64), lambda i: (0, 0)),
            pl.BlockSpec((8, 64), lambda i: (0, 0)),
            pl.BlockSpec((1, 64), lambda i: (0, 0)),
        ],
        out_specs=[
            pl.BlockSpec((_BLK, 64), lambda i: (i, 0)),
            pl.BlockSpec((_BLK, _HF), lambda i: (i, 0)),
            pl.BlockSpec((_BLK, _HF), lambda i: (i, 0)),
        ],
        out_shape=[
            jax.ShapeDtypeStruct((_NPAD, 64), jnp.float32),
            jax.ShapeDtypeStruct((_NPAD, _HF), jnp.float32),
            jax.ShapeDtypeStruct((_NPAD, _HF), jnp.float32),
        ],
    )(x8, w1t, w1b, b1)


def _combine(a, m0, m1):
    mm = jnp.concatenate([m0, m1], axis=1)
    return jnp.where(mm > _NEG * 0.5, jnp.maximum(a + mm, 0.0), 0.0)


def _cab_body(a_ref, m0_ref, m1_ref, w_ref, b_ref, aout_ref, b0_ref, b1_ref):
    xb = _combine(a_ref[...], m0_ref[...], m1_ref[...])
    wb = w_ref[64:128, :]
    wa = w_ref[0:64, :] - wb
    aout_ref[...] = (
        jnp.dot(xb, wa, preferred_element_type=jnp.float32) + b_ref[...]
    )
    bb = jnp.dot(xb, wb, preferred_element_type=jnp.float32)
    b0_ref[...] = bb[:, :_HF]
    b1_ref[...] = bb[:, _HF:]


def _cab_layer(a_prev, m0, m1, w, b):
    return pl.pallas_call(
        _cab_body,
        grid=(_NBLK,),
        in_specs=[
            pl.BlockSpec((_BLK, 64), lambda i: (i, 0)),
            pl.BlockSpec((_BLK, _HF), lambda i: (i, 0)),
            pl.BlockSpec((_BLK, _HF), lambda i: (i, 0)),
            pl.BlockSpec((128, 64), lambda i: (0, 0)),
            pl.BlockSpec((1, 64), lambda i: (0, 0)),
        ],
        out_specs=[
            pl.BlockSpec((_BLK, 64), lambda i: (i, 0)),
            pl.BlockSpec((_BLK, _HF), lambda i: (i, 0)),
            pl.BlockSpec((_BLK, _HF), lambda i: (i, 0)),
        ],
        out_shape=[
            jax.ShapeDtypeStruct((_NPAD, 64), jnp.float32),
            jax.ShapeDtypeStruct((_NPAD, _HF), jnp.float32),
            jax.ShapeDtypeStruct((_NPAD, _HF), jnp.float32),
        ],
    )(a_prev, m0, m1, w, b)


def _x5g_body(a_ref, m0_ref, m1_ref, x5_ref, g_ref):
    x5 = _combine(a_ref[...], m0_ref[...], m1_ref[...])
    x5_ref[...] = x5
    blk = jnp.max(x5, axis=0, keepdims=True)

    @pl.when(pl.program_id(0) == 0)
    def _():
        g_ref[...] = blk

    @pl.when(pl.program_id(0) > 0)
    def _():
        g_ref[...] = jnp.maximum(g_ref[...], blk)


def _x5g(a5, m0, m1):
    return pl.pallas_call(
        _x5g_body,
        grid=(_NBLK,),
        in_specs=[
            pl.BlockSpec((_BLK, 64), lambda i: (i, 0)),
            pl.BlockSpec((_BLK, _HF), lambda i: (i, 0)),
            pl.BlockSpec((_BLK, _HF), lambda i: (i, 0)),
        ],
        out_specs=[
            pl.BlockSpec((_BLK, 64), lambda i: (i, 0)),
            pl.BlockSpec((1, 64), lambda i: (0, 0)),
        ],
        out_shape=[
            jax.ShapeDtypeStruct((_NPAD, 64), jnp.float32),
            jax.ShapeDtypeStruct((1, 64), jnp.float32),
        ],
    )(a5, m0, m1)


def _mlp_body(x5_ref, g_ref, x8_ref, wl1t_ref, wl1b_ref, bl1_ref,
              wl2_ref, bl2_ref, out_ref):
    h = jnp.maximum(
        jnp.dot(x5_ref[...], wl1t_ref[...], preferred_element_type=jnp.float32)
        + jnp.dot(g_ref[...], wl1b_ref[...], preferred_element_type=jnp.float32)
        + bl1_ref[...],
        0.0,
    )
    out_ref[...] = (
        x8_ref[...]
        + jnp.dot(h, wl2_ref[...], preferred_element_type=jnp.float32)
        + bl2_ref[...]
    )


def _mlp(x5, g, x8, wl1t, wl1b, bl1, wl2p, bl2p):
    return pl.pallas_call(
        _mlp_body,
        grid=(_NBLK,),
        in_specs=[
            pl.BlockSpec((_BLK, 64), lambda i: (i, 0)),
            pl.BlockSpec((1, 64), lambda i: (0, 0)),
            pl.BlockSpec((_BLK, 8), lambda i: (i, 0)),
            pl.BlockSpec((64, 128), lambda i: (0, 0)),
            pl.BlockSpec((64, 128), lambda i: (0, 0)),
            pl.BlockSpec((1, 128), lambda i: (0, 0)),
            pl.BlockSpec((128, 8), lambda i: (0, 0)),
            pl.BlockSpec((1, 8), lambda i: (0, 0)),
        ],
        out_specs=pl.BlockSpec((_BLK, 8), lambda i: (i, 0)),
        out_shape=jax.ShapeDtypeStruct((_NPAD, 8), jnp.float32),
    )(x5, g, x8, wl1t, wl1b, bl1, wl2p, bl2p)


def kernel(x, edge_index, W1, b1, W2, b2, W3, b3, W4, b4, W5, b5,
           Wl1, bl1, Wl2, bl2):
    src = edge_index[0]
    dst = edge_index[1]
    x8 = jnp.zeros((_NPAD, 8), jnp.float32).at[:_N, :3].set(x)
    w1t = jnp.zeros((8, 64), jnp.float32).at[:3].set(W1[:3])
    w1b = jnp.zeros((8, 64), jnp.float32).at[:3].set(W1[3:])

    lsrc, lld, cnt = _bucketize(src, dst)

    A, B0, B1 = _ab_layer1(x8, w1t, w1b, b1.reshape(1, 64))
    M0, M1 = _segmax(B0, B1, lsrc, lld, cnt)
    for w, b in ((W2, b2), (W3, b3), (W4, b4), (W5, b5)):
        A, B0, B1 = _cab_layer(A, M0, M1, w, b.reshape(1, 64))
        M0, M1 = _segmax(B0, B1, lsrc, lld, cnt)

    x5, g = _x5g(A, M0, M1)
    wl2p = jnp.zeros((128, 8), jnp.float32).at[:, :3].set(Wl2)
    bl2p = jnp.zeros((1, 8), jnp.float32).at[0, :3].set(bl2)
    out = _mlp(x5, g, x8, Wl1[:64], Wl1[64:], bl1.reshape(1, 128),
               wl2p, bl2p)
    return out[:_N, :3]


# speedup vs baseline: 4.0636x; 1.0925x over previous
import functools

import jax
import jax.numpy as jnp
from jax import lax
from jax.experimental import pallas as pl
from jax.experimental.pallas import tpu as pltpu
from jax.experimental.pallas import tpu_sc as plsc

_N = 50000
_E = 800000
_NW = 32
_RPW = 1600
_NPAD = _NW * _RPW
_NEG = -3.0e38
_C = 1600
_NCHUNK = _E // _C
_VPC = _C // 16
_F = 512
_ECAP = (_E // _F + 2) * _F
_G = 128
_HF = 32
_ACC_ROWS = _RPW + 2

_BLK = 128
_NBLK = _NPAD // _BLK

_sc_mesh = plsc.VectorSubcoreMesh(core_axis_name="c", subcore_axis_name="s")


@functools.partial(
    pl.kernel,
    out_type=[
        jax.ShapeDtypeStruct((_NW * _ECAP,), jnp.int32),
        jax.ShapeDtypeStruct((_NW * _ECAP,), jnp.int32),
        jax.ShapeDtypeStruct((_NW * 16,), jnp.int32),
    ],
    mesh=_sc_mesh,
    scratch_types=[
        pltpu.VMEM((_C,), jnp.int32),
        pltpu.VMEM((_C,), jnp.int32),
        pltpu.VMEM((_C,), jnp.int32),
        pltpu.VMEM((_C,), jnp.int32),
        pltpu.VMEM((_F + _C + 16,), jnp.int32),
        pltpu.VMEM((_F + _C + 16,), jnp.int32),
        pltpu.VMEM((16,), jnp.int32),
        pltpu.SemaphoreType.DMA,
        pltpu.SemaphoreType.DMA,
        pltpu.SemaphoreType.DMA,
        pltpu.SemaphoreType.DMA,
        pltpu.SemaphoreType.DMA,
    ],
    compiler_params=pltpu.CompilerParams(needs_layout_passes=False),
)
def _bucketize(src_hbm, dst_hbm, lsrc_hbm, lld_hbm, cnt_hbm,
               dstb0, dstb1, srcb0, srcb1, sbs, sbl, cntv,
               sd0, sd1, ss0, ss1, so):
    cidx = lax.axis_index("c")
    sidx = lax.axis_index("s")
    wid = sidx * 2 + cidx
    lo = wid * _RPW
    lanes = lax.iota(jnp.int32, 16)
    pad16 = jnp.full((16,), _RPW, jnp.int32)

    def _issue(k, dref, sref, dsem, ssem):
        pltpu.make_async_copy(dst_hbm.at[pl.ds(k * _C, _C)], dref, dsem).start()
        pltpu.make_async_copy(src_hbm.at[pl.ds(k * _C, _C)], sref, ssem).start()

    def _wait(k, dref, sref, dsem, ssem):
        pltpu.make_async_copy(dst_hbm.at[pl.ds(k * _C, _C)], dref, dsem).wait()
        pltpu.make_async_copy(src_hbm.at[pl.ds(k * _C, _C)], sref, ssem).wait()

    def _scan(dref, sref, pending):
        def scan_body(v, pending):
            d = dref[pl.ds(v * 16, 16)]
            s = sref[pl.ds(v * 16, 16)]
            m = (d >= lo) & (d < lo + _RPW)
            cum = plsc.cumsum(m.astype(jnp.int32))
            pos = pending + cum - 1
            plsc.store_scatter(sbs, [pos], s, mask=m)
            plsc.store_scatter(sbl, [pos], d - lo, mask=m)
            return pending + cum[15]

        return lax.fori_loop(0, _VPC, scan_body, pending, unroll=2)

    def _flush(state):
        pending, flushed = state
        cp = pltpu.make_async_copy(
            sbs.at[pl.ds(0, _F)],
            lsrc_hbm.at[pl.ds(pl.multiple_of(wid * _ECAP + flushed, 8), _F)],
            so)
        cp.start()
        cp.wait()
        cp = pltpu.make_async_copy(
            sbl.at[pl.ds(0, _F)],
            lld_hbm.at[pl.ds(pl.multiple_of(wid * _ECAP + flushed, 8), _F)],
            so)
        cp.start()
        cp.wait()

        def shift(v, carry):
            sbs[pl.ds(v * 16, 16)] = sbs[pl.ds(_F + v * 16, 16)]
            sbl[pl.ds(v * 16, 16)] = sbl[pl.ds(_F + v * 16, 16)]
            return carry

        lax.fori_loop(0, (_C + 16) // 16, shift, 0, unroll=2)
        return pending - _F, flushed + _F

    def chunk_pair(i, state):
        k0 = i * 2
        state = _drain_scan(k0, dstb0, srcb0, sd0, ss0, state)

        @pl.when(k0 + 2 < _NCHUNK)
        def _():
            _issue(k0 + 2, dstb0, srcb0, sd0, ss0)

        state = _drain_scan(k0 + 1, dstb1, srcb1, sd1, ss1, state)

        @pl.when(k0 + 3 < _NCHUNK)
        def _():
            _issue(k0 + 3, dstb1, srcb1, sd1, ss1)

        return state

    def _drain_scan(k, dref, sref, dsem, ssem, state):
        pending, flushed = state
        _wait(k, dref, sref, dsem, ssem)
        pending = _scan(dref, sref, pending)
        return lax.while_loop(
            lambda st: st[0] >= _F, _flush, (pending, flushed))

    _issue(0, dstb0, srcb0, sd0, ss0)
    _issue(1, dstb1, srcb1, sd1, ss1)
    pending, flushed = lax.fori_loop(
        0, _NCHUNK // 2, chunk_pair, (jnp.int32(0), jnp.int32(0)))

    for j in range(_F // 16):
        pos = pending + lanes + j * 16
        plsc.store_scatter(sbs, [pos], lo + lanes + j * 16)
        plsc.store_scatter(sbl, [pos], pad16)
    _flush((pending, flushed))
    total = flushed + _F

    cntv[...] = jnp.where(lanes == 0, total, 0)
    pltpu.sync_copy(cntv, cnt_hbm.at[pl.ds(pl.multiple_of(wid * 16, 8), 16)])


@functools.partial(
    pl.kernel,
    out_type=jax.ShapeDtypeStruct((_NPAD * _HF,), jnp.float32),
    mesh=_sc_mesh,
    scratch_types=[
        pltpu.VMEM((_G,), jnp.int32),
        pltpu.VMEM((_G,), jnp.int32),
        pltpu.VMEM((_G,), jnp.int32),
        pltpu.VMEM((_G,), jnp.int32),
        pltpu.VMEM((_G, _HF), jnp.float32),
        pltpu.VMEM((_G, _HF), jnp.float32),
        pltpu.VMEM((_ACC_ROWS * _HF,), jnp.float32),
        pltpu.VMEM((16,), jnp.int32),
        pltpu.SemaphoreType.DMA,
        pltpu.SemaphoreType.DMA,
        pltpu.SemaphoreType.DMA,
        pltpu.SemaphoreType.DMA,
        pltpu.SemaphoreType.DMA,
        pltpu.SemaphoreType.DMA,
        pltpu.SemaphoreType.DMA,
    ],
    compiler_params=pltpu.CompilerParams(use_tc_tiling_on_sc=False),
)
def _segmax_half(bh_hbm, lsrc_hbm, lld_hbm, cnt_hbm, mh_hbm,
                 gidx0, gidx1, lldv0, lldv1, rows0, rows1, acc, cntv,
                 sa0, sa1, sb0, sb1, sg0, sg1, sc):
    cidx = lax.axis_index("c")
    sidx = lax.axis_index("s")
    wid = sidx * 2 + cidx
    lo = wid * _RPW
    neg16 = jnp.full((16,), _NEG, jnp.float32)

    def _init(i, carry):
        acc[pl.ds(i * 16, 16)] = neg16
        return carry

    lax.fori_loop(0, _ACC_ROWS * _HF // 16, _init, 0, unroll=4)

    cp = pltpu.make_async_copy(
        cnt_hbm.at[pl.ds(pl.multiple_of(wid * 16, 8), 16)], cntv, sc)
    cp.start()
    cp.wait()
    n = cntv[...][0]
    nb = n // _G

    def _loff(b):
        return pl.ds(pl.multiple_of(wid * _ECAP + b * _G, 8), _G)

    def _list_start(b, gi, lv, sa, sb):
        pltpu.make_async_copy(lsrc_hbm.at[_loff(b)], gi, sa).start()
        pltpu.make_async_copy(lld_hbm.at[_loff(b)], lv, sb).start()

    def _list_wait(b, gi, lv, sa, sb):
        pltpu.make_async_copy(lsrc_hbm.at[_loff(b)], gi, sa).wait()
        pltpu.make_async_copy(lld_hbm.at[_loff(b)], lv, sb).wait()

    def _apply(lv, rw):
        for q in range(_G // 16):
            ldv = lv[pl.ds(q * 16, 16)]
            for lane in range(16):
                base = pl.multiple_of(ldv[lane] * _HF, 8)
                for cg in range(_HF // 16):
                    cur = acc[pl.ds(base + cg * 16, 16)]
                    new = rw[q * 16 + lane, pl.ds(cg * 16, 16)]
                    acc[pl.ds(base + cg * 16, 16)] = jnp.maximum(cur, new)

    _list_start(0, gidx0, lldv0, sa0, sb0)
    _list_start(1, gidx1, lldv1, sa1, sb1)
    _list_wait(0, gidx0, lldv0, sa0, sb0)
    pltpu.make_async_copy(bh_hbm.at[gidx0], rows0, sg0).start()

    def pair_body(i, carry):
        b0 = i * 2
        b1 = b0 + 1
        _list_wait(b1, gidx1, lldv1, sa1, sb1)
        pltpu.make_async_copy(bh_hbm.at[gidx1], rows1, sg1).start()
        pltpu.make_async_copy(bh_hbm.at[gidx0], rows0, sg0).wait()
        _apply(lldv0, rows0)

        @pl.when(b0 + 2 < nb)
        def _():
            _list_start(b0 + 2, gidx0, lldv0, sa0, sb0)

        pltpu.make_async_copy(bh_hbm.at[gidx1], rows1, sg1).wait()

        @pl.when(b0 + 2 < nb)
        def _():
            _list_wait(b0 + 2, gidx0, lldv0, sa0, sb0)
            pltpu.make_async_copy(bh_hbm.at[gidx0], rows0, sg0).start()

        _apply(lldv1, rows1)

        @pl.when(b1 + 2 < nb)
        def _():
            _list_start(b1 + 2, gidx1, lldv1, sa1, sb1)

        return carry

    lax.fori_loop(0, nb // 2, pair_body, 0)

    pltpu.sync_copy(acc.at[pl.ds(0, _RPW * _HF)],
                    mh_hbm.at[pl.ds(pl.multiple_of(lo * _HF, 8), _RPW * _HF)])


def _segmax(b0, b1, lsrc, lld, cnt):
    m0 = _segmax_half(b0, lsrc, lld, cnt).reshape(_NPAD, _HF)
    m1 = _segmax_half(b1, lsrc, lld, cnt).reshape(_NPAD, _HF)
    return m0, m1


def _ab_body(x_ref, wt_ref, wb_ref, b_ref, a_ref, b0_ref, b1_ref):
    xb = x_ref[...]
    wb = wb_ref[...]
    wa = wt_ref[...] - wb
    a_ref[...] = (
        jnp.dot(xb, wa, preferred_element_type=jnp.float32) + b_ref[...]
    )
    bb = jnp.dot(xb, wb, preferred_element_type=jnp.float32)
    b0_ref[...] = bb[:, :_HF]
    b1_ref[...] = bb[:, _HF:]


def _ab_layer1(x8, w1t, w1b, b1):
    return pl.pallas_call(
        _ab_body,
        grid=(_NBLK,),
        in_specs=[
            pl.BlockSpec((_BLK, 8), lambda i: (i, 0)),
            pl.BlockSpec((8, 64), lambda i: (0, 0)),
            pl.BlockSpec((8, 64), lambda i: (0, 0)),
            pl.BlockSpec((1, 64), lambda i: (0, 0)),
        ],
        out_specs=[
            pl.BlockSpec((_BLK, 64), lambda i: (i, 0)),
            pl.BlockSpec((_BLK, _HF), lambda i: (i, 0)),
            pl.BlockSpec((_BLK, _HF), lambda i: (i, 0)),
        ],
        out_shape=[
            jax.ShapeDtypeStruct((_NPAD, 64), jnp.float32),
            jax.ShapeDtypeStruct((_NPAD, _HF), jnp.float32),
            jax.ShapeDtypeStruct((_NPAD, _HF), jnp.float32),
        ],
    )(x8, w1t, w1b, b1)


def _combine(a, m0, m1):
    mm = jnp.concatenate([m0, m1], axis=1)
    return jnp.where(mm > _NEG * 0.5, jnp.maximum(a + mm, 0.0), 0.0)


def _cab_body(a_ref, m0_ref, m1_ref, w_ref, b_ref, aout_ref, b0_ref, b1_ref):
    xb = _combine(a_ref[...], m0_ref[...], m1_ref[...])
    wb = w_ref[64:128, :]
    wa = w_ref[0:64, :] - wb
    aout_ref[...] = (
        jnp.dot(xb, wa, preferred_element_type=jnp.float32) + b_ref[...]
    )
    bb = jnp.dot(xb, wb, preferred_element_type=jnp.float32)
    b0_ref[...] = bb[:, :_HF]
    b1_ref[...] = bb[:, _HF:]


def _cab_layer(a_prev, m0, m1, w, b):
    return pl.pallas_call(
        _cab_body,
        grid=(_NBLK,),
        in_specs=[
            pl.BlockSpec((_BLK, 64), lambda i: (i, 0)),
            pl.BlockSpec((_BLK, _HF), lambda i: (i, 0)),
            pl.BlockSpec((_BLK, _HF), lambda i: (i, 0)),
            pl.BlockSpec((128, 64), lambda i: (0, 0)),
            pl.BlockSpec((1, 64), lambda i: (0, 0)),
        ],
        out_specs=[
            pl.BlockSpec((_BLK, 64), lambda i: (i, 0)),
            pl.BlockSpec((_BLK, _HF), lambda i: (i, 0)),
            pl.BlockSpec((_BLK, _HF), lambda i: (i, 0)),
        ],
        out_shape=[
            jax.ShapeDtypeStruct((_NPAD, 64), jnp.float32),
            jax.ShapeDtypeStruct((_NPAD, _HF), jnp.float32),
            jax.ShapeDtypeStruct((_NPAD, _HF), jnp.float32),
        ],
    )(a_prev, m0, m1, w, b)


def _x5g_body(a_ref, m0_ref, m1_ref, x5_ref, g_ref):
    x5 = _combine(a_ref[...], m0_ref[...], m1_ref[...])
    x5_ref[...] = x5
    blk = jnp.max(x5, axis=0, keepdims=True)

    @pl.when(pl.program_id(0) == 0)
    def _():
        g_ref[...] = blk

    @pl.when(pl.program_id(0) > 0)
    def _():
        g_ref[...] = jnp.maximum(g_ref[...], blk)


def _x5g(a5, m0, m1):
    return pl.pallas_call(
        _x5g_body,
        grid=(_NBLK,),
        in_specs=[
            pl.BlockSpec((_BLK, 64), lambda i: (i, 0)),
            pl.BlockSpec((_BLK, _HF), lambda i: (i, 0)),
            pl.BlockSpec((_BLK, _HF), lambda i: (i, 0)),
        ],
        out_specs=[
            pl.BlockSpec((_BLK, 64), lambda i: (i, 0)),
            pl.BlockSpec((1, 64), lambda i: (0, 0)),
        ],
        out_shape=[
            jax.ShapeDtypeStruct((_NPAD, 64), jnp.float32),
            jax.ShapeDtypeStruct((1, 64), jnp.float32),
        ],
    )(a5, m0, m1)


def _mlp_body(x5_ref, g_ref, x8_ref, wl1t_ref, wl1b_ref, bl1_ref,
              wl2_ref, bl2_ref, out_ref):
    h = jnp.maximum(
        jnp.dot(x5_ref[...], wl1t_ref[...], preferred_element_type=jnp.float32)
        + jnp.dot(g_ref[...], wl1b_ref[...], preferred_element_type=jnp.float32)
        + bl1_ref[...],
        0.0,
    )
    out_ref[...] = (
        x8_ref[...]
        + jnp.dot(h, wl2_ref[...], preferred_element_type=jnp.float32)
        + bl2_ref[...]
    )


def _mlp(x5, g, x8, wl1t, wl1b, bl1, wl2p, bl2p):
    return pl.pallas_call(
        _mlp_body,
        grid=(_NBLK,),
        in_specs=[
            pl.BlockSpec((_BLK, 64), lambda i: (i, 0)),
            pl.BlockSpec((1, 64), lambda i: (0, 0)),
            pl.BlockSpec((_BLK, 8), lambda i: (i, 0)),
            pl.BlockSpec((64, 128), lambda i: (0, 0)),
            pl.BlockSpec((64, 128), lambda i: (0, 0)),
            pl.BlockSpec((1, 128), lambda i: (0, 0)),
            pl.BlockSpec((128, 8), lambda i: (0, 0)),
            pl.BlockSpec((1, 8), lambda i: (0, 0)),
        ],
        out_specs=pl.BlockSpec((_BLK, 8), lambda i: (i, 0)),
        out_shape=jax.ShapeDtypeStruct((_NPAD, 8), jnp.float32),
    )(x5, g, x8, wl1t, wl1b, bl1, wl2p, bl2p)


def kernel(x, edge_index, W1, b1, W2, b2, W3, b3, W4, b4, W5, b5,
           Wl1, bl1, Wl2, bl2):
    src = edge_index[0]
    dst = edge_index[1]
    x8 = jnp.zeros((_NPAD, 8), jnp.float32).at[:_N, :3].set(x)
    w1t = jnp.zeros((8, 64), jnp.float32).at[:3].set(W1[:3])
    w1b = jnp.zeros((8, 64), jnp.float32).at[:3].set(W1[3:])

    lsrc, lld, cnt = _bucketize(src, dst)

    A, B0, B1 = _ab_layer1(x8, w1t, w1b, b1.reshape(1, 64))
    M0, M1 = _segmax(B0, B1, lsrc, lld, cnt)
    for w, b in ((W2, b2), (W3, b3), (W4, b4), (W5, b5)):
        A, B0, B1 = _cab_layer(A, M0, M1, w, b.reshape(1, 64))
        M0, M1 = _segmax(B0, B1, lsrc, lld, cnt)

    x5, g = _x5g(A, M0, M1)
    wl2p = jnp.zeros((128, 8), jnp.float32).at[:, :3].set(Wl2)
    bl2p = jnp.zeros((1, 8), jnp.float32).at[0, :3].set(bl2)
    out = _mlp(x5, g, x8, Wl1[:64], Wl1[64:], bl1.reshape(1, 128),
               wl2p, bl2p)
    return out[:_N, :3]
